# baseline (speedup 1.0000x reference)
import functools

import jax
import jax.numpy as jnp
import numpy as np
from jax import lax
from jax.experimental import pallas as pl
from jax.experimental.pallas import tpu as pltpu
from jax.experimental.pallas import tpu_sc as plsc

F32 = jnp.float32

_TAP = ((0, 1), (1, 0), (1, 1), (2, 0))


def _c1_body(xp_ref, w_ref, b_ref, o_ref):
    parts = []
    for dt in (0, 1):
        row = []
        for dv in (0, 1):
            row.append(xp_ref[0, dt:dt + 56, dv:dv + 56, :])
        parts.append(jnp.concatenate(row, axis=2))
    xs = jnp.concatenate(parts, axis=2).reshape(3136, 256)
    acc = jnp.dot(xs, w_ref[...], preferred_element_type=F32)
    h = jnp.maximum(acc + b_ref[...], 0.0)
    o_ref[0, 1:57, 1:57, :] = h.reshape(56, 56, 1024)
    o_ref[0, 0:1, :, :] = jnp.zeros((1, 58, 1024), F32)
    o_ref[0, 57:58, :, :] = jnp.zeros((1, 58, 1024), F32)
    o_ref[0, 1:57, 0:1, :] = jnp.zeros((56, 1, 1024), F32)
    o_ref[0, 1:57, 57:58, :] = jnp.zeros((56, 1, 1024), F32)


def _conv1(xp, w, b):
    return pl.pallas_call(
        _c1_body,
        grid=(8,),
        in_specs=[
            pl.BlockSpec((1, 57, 57, 64), lambda i: (i, 0, 0, 0)),
            pl.BlockSpec((256, 1024), lambda i: (0, 0)),
            pl.BlockSpec((1, 1024), lambda i: (0, 0)),
        ],
        out_specs=pl.BlockSpec((1, 58, 58, 1024), lambda i: (i, 0, 0, 0)),
        out_shape=jax.ShapeDtypeStruct((8, 58, 58, 1024), F32),
    )(xp, w, b)


def _c2_body(hs_ref, w_ref, ct_ref, c2_ref, b_ref, idx_ref, s_ref):
    i = pl.program_id(0)
    ct = ct_ref[...]
    acc = jnp.zeros((3136, 256), F32)
    for ky in range(4):
        oR, p = _TAP[ky]
        for kx in range(4):
            oC, q = _TAP[kx]
            lo = 512 * p + 256 * q
            xs = hs_ref[0, oR:oR + 56, oC:oC + 56, lo:lo + 256]
            acc = acc + jnp.dot(xs.reshape(3136, 256), w_ref[ky, kx],
                                preferred_element_type=F32)
    z = acc + b_ref[...]
    part = jnp.zeros((1, 1), F32)
    for hh in (0, 1568):
        zh = z[hh:hh + 1568, :]
        z2 = jnp.sum(zh * zh, axis=1, keepdims=True)
        mm = jnp.dot(zh, ct, preferred_element_type=F32)
        scores = (z2 - 2.0 * mm) + c2_ref[...]
        mins = jnp.min(scores, axis=1, keepdims=True)
        iota = lax.broadcasted_iota(jnp.int32, scores.shape, 1)
        idx = jnp.min(jnp.where(scores == mins, iota, jnp.int32(2 ** 30)),
                      axis=1)
        idx_ref[0, 0, pl.ds(hh, 1568)] = idx
        part = part + jnp.sum(mins).reshape(1, 1)

    @pl.when(i == 0)
    def _():
        s_ref[...] = part

    @pl.when(i > 0)
    def _():
        s_ref[...] += part


def _conv2vq(hs, w, ct, c2, b):
    idx, qsum = pl.pallas_call(
        _c2_body,
        grid=(8,),
        in_specs=[
            pl.BlockSpec((1, 58, 58, 1024), lambda i: (i, 0, 0, 0)),
            pl.BlockSpec((4, 4, 256, 256), lambda i: (0, 0, 0, 0)),
            pl.BlockSpec((256, 1024), lambda i: (0, 0)),
            pl.BlockSpec((1, 1024), lambda i: (0, 0)),
            pl.BlockSpec((1, 256), lambda i: (0, 0)),
        ],
        out_specs=[
            pl.BlockSpec((1, 1, 3136), lambda i: (i, 0, 0)),
            pl.BlockSpec((1, 1), lambda i: (0, 0)),
        ],
        out_shape=[
            jax.ShapeDtypeStruct((8, 1, 3136), jnp.int32),
            jax.ShapeDtypeStruct((1, 1), F32),
        ],
    )(hs, w, ct, c2, b)
    return idx.reshape(25088), qsum[0, 0]


@functools.lru_cache(maxsize=None)
def _make_sc_gather():
    mesh = plsc.VectorSubcoreMesh(core_axis_name="c", subcore_axis_name="s")

    @functools.partial(
        pl.kernel, mesh=mesh,
        out_type=jax.ShapeDtypeStruct((25088, 256), F32),
        scratch_types=[
            pltpu.VMEM((784,), jnp.int32),
            pltpu.VMEM((112, 256), F32),
            pltpu.VMEM((112, 256), F32),
            pltpu.SemaphoreType.DMA,
            pltpu.SemaphoreType.DMA,
            pltpu.SemaphoreType.DMA,
            pltpu.SemaphoreType.DMA,
        ],
    )
    def _gather(table_hbm, idx_hbm, out_hbm, idx_v, r0, r1, g0, g1, o0, o1):
        wid = lax.axis_index("s") * 2 + lax.axis_index("c")
        base = wid * 784
        pltpu.sync_copy(idx_hbm.at[pl.ds(base, 784)], idx_v)
        bufs = (r0, r1)
        gsem = (g0, g1)
        osem = (o0, o1)
        gcp = [None] * 7
        ocp = [None] * 7
        gcp[0] = pltpu.async_copy(
            table_hbm.at[idx_v.at[pl.ds(0, 112)]], bufs[0], gsem[0])
        for c in range(7):
            if c + 1 < 7:
                if c - 1 >= 0:
                    ocp[c - 1].wait()
                gcp[c + 1] = pltpu.async_copy(
                    table_hbm.at[idx_v.at[pl.ds(112 * (c + 1), 112)]],
                    bufs[(c + 1) % 2], gsem[(c + 1) % 2])
            gcp[c].wait()
            ocp[c] = pltpu.async_copy(
                bufs[c % 2], out_hbm.at[pl.ds(base + 112 * c, 112)],
                osem[c % 2])
        ocp[5].wait()
        ocp[6].wait()

    return _gather


def _sc_gather(table, idx):
    return _make_sc_gather()(table, idx)


def _dct1_body(q_ref, w_ref, b_ref, o_ref, qp_ref):
    i = pl.program_id(0)

    @pl.when(i == 0)
    def _():
        qp_ref[...] = jnp.zeros((58, 58, 256), F32)

    qp_ref[1:57, 1:57, :] = q_ref[0]
    outs = []
    for r in (0, 1):
        for s in (0, 1):
            acc = jnp.zeros((3136, 256), F32)
            for m in (0, 1):
                for n in (0, 1):
                    xs = qp_ref[r + m:r + m + 56, s + n:s + n + 56, :]
                    acc = acc + jnp.dot(xs.reshape(3136, 256),
                                        w_ref[r + 2 * m, s + 2 * n],
                                        preferred_element_type=F32)
            outs.append(acc)
    out = jnp.concatenate(outs, axis=1) + b_ref[...]
    o_ref[0] = jnp.maximum(out, 0.0).reshape(56, 56, 1024)


def _dct1(q, w, b):
    return pl.pallas_call(
        _dct1_body,
        grid=(8,),
        in_specs=[
            pl.BlockSpec((1, 56, 56, 256), lambda i: (i, 0, 0, 0)),
            pl.BlockSpec((4, 4, 256, 256), lambda i: (0, 0, 0, 0)),
            pl.BlockSpec((1, 1024), lambda i: (0, 0)),
        ],
        out_specs=pl.BlockSpec((1, 56, 56, 1024), lambda i: (i, 0, 0, 0)),
        out_shape=jax.ShapeDtypeStruct((8, 56, 56, 1024), F32),
        scratch_shapes=[pltpu.VMEM((58, 58, 256), F32)],
    )(q, w, b)


def _dct2_body(d_ref, w_ref, b_ref, o_ref, dp_ref):
    i = pl.program_id(0)

    @pl.when(i == 0)
    def _():
        dp_ref[...] = jnp.zeros((58, 58, 1024), F32)

    dp_ref[1:57, 1:57, :] = d_ref[0]
    for ho in (0, 28):
        accs = []
        for pi in (0, 1):
            for chi in (0, 1):
                acc = jnp.broadcast_to(b_ref[...], (1568, 16))
                for u in range(3):
                    eu = pi + u - 1
                    oA, rho = (eu + 2) // 2, (eu + 2) % 2
                    for v in range(3):
                        ev = chi + v - 1
                        oB, sig = (ev + 2) // 2, (ev + 2) % 2
                        lo = (rho * 2 + sig) * 256
                        xs = dp_ref[ho + oA:ho + oA + 28, oB:oB + 56,
                                    lo:lo + 256]
                        acc = acc + jnp.dot(xs.reshape(1568, 256),
                                            w_ref[3 * u + v],
                                            preferred_element_type=F32)
                accs.append(acc)
        out = jnp.concatenate(accs, axis=1)
        o_ref[0, pl.ds(ho, 28)] = out.reshape(28, 56, 64)


def _dct2(dph, w9, b16):
    return pl.pallas_call(
        _dct2_body,
        grid=(8,),
        in_specs=[
            pl.BlockSpec((1, 56, 56, 1024), lambda i: (i, 0, 0, 0)),
            pl.BlockSpec((9, 256, 16), lambda i: (0, 0, 0)),
            pl.BlockSpec((1, 16), lambda i: (0, 0)),
        ],
        out_specs=pl.BlockSpec((1, 56, 56, 64), lambda i: (i, 0, 0, 0)),
        out_shape=jax.ShapeDtypeStruct((8, 56, 56, 64), F32),
        scratch_shapes=[pltpu.VMEM((58, 58, 1024), F32)],
    )(dph, w9, b16)


def _loss_body(a_ref, b_ref, s_ref):
    i = pl.program_id(0)
    diff = a_ref[...] - b_ref[...]
    part = jnp.sum(diff * diff).reshape(1, 1)

    @pl.when(i == 0)
    def _():
        s_ref[...] = part

    @pl.when(i > 0)
    def _():
        s_ref[...] += part


def _sqdiff_sum(a, b):
    af = a.reshape(9408, 128)
    bf = b.reshape(9408, 128)
    out = pl.pallas_call(
        _loss_body,
        grid=(8,),
        in_specs=[
            pl.BlockSpec((1176, 128), lambda i: (i, 0)),
            pl.BlockSpec((1176, 128), lambda i: (i, 0)),
        ],
        out_specs=pl.BlockSpec((1, 1), lambda i: (0, 0)),
        out_shape=jax.ShapeDtypeStruct((1, 1), F32),
    )(af, bf)
    return out[0, 0]


def kernel(x, enc_w1, enc_b1, enc_w2, enc_b2, codebook, dec_w1, dec_b1,
           dec_w2, dec_b2):
    B = 8

    xr = jnp.pad(x, ((0, 0), (0, 0), (1, 3), (1, 3)))
    xs2d = xr.reshape(B, 3, 57, 4, 57, 4).transpose(0, 2, 4, 3, 5, 1)
    xs2d = jnp.pad(xs2d, ((0, 0),) * 5 + ((0, 1),))
    xs2d = xs2d.reshape(B, 57, 57, 64)
    w1t = enc_w1.transpose(2, 3, 1, 0)
    ky_i = np.zeros((256, 4), np.int32)
    kx_i = np.zeros((256, 4), np.int32)
    c_i = np.zeros((256, 4), np.int32)
    val = np.zeros((256, 4), np.float32)
    for dt in range(2):
        for dv in range(2):
            for j in range(4):
                for k in range(4):
                    for c4 in range(4):
                        r = ((dt * 2 + dv) * 16 + j * 4 + k) * 4 + c4
                        for P in range(2):
                            for Q in range(2):
                                g = P * 2 + Q
                                ky = 4 * dt + j - 2 * P
                                kx = 4 * dv + k - 2 * Q
                                if 0 <= ky <= 3 and 0 <= kx <= 3 and c4 < 3:
                                    ky_i[r, g], kx_i[r, g], c_i[r, g] = ky, kx, c4
                                    val[r, g] = 1.0
    w1p = (w1t[ky_i, kx_i, c_i, :] * val[:, :, None]).reshape(256, 1024)
    b1s = jnp.tile(enc_b1, 4).reshape(1, 1024)
    hs = _conv1(xs2d, w1p, b1s)

    w2t = enc_w2.transpose(2, 3, 1, 0)
    ct = codebook.T
    c2 = jnp.sum(codebook ** 2, axis=1).reshape(1, 1024)
    idx, qsum = _conv2vq(hs, w2t, ct, c2, enc_b2.reshape(1, 256))

    quantized = _sc_gather(codebook, idx)
    q = quantized.reshape(B, 56, 56, 256)

    w3t = dec_w1.transpose(2, 3, 1, 0)
    b3 = jnp.tile(dec_b1, 4).reshape(1, 1024)
    dph = _dct1(q, w3t, b3)

    w4t = dec_w2.transpose(2, 3, 1, 0)
    ky9 = np.zeros((9, 4), np.int32)
    kx9 = np.zeros((9, 4), np.int32)
    v9 = np.zeros((9, 4), np.float32)
    for u in range(3):
        for v in range(3):
            for r in range(2):
                for s in range(2):
                    if 0 <= u - r <= 1 and 0 <= v - s <= 1:
                        ky9[3 * u + v, r * 2 + s] = 2 * u - r
                        kx9[3 * u + v, r * 2 + s] = 2 * v - s
                        v9[3 * u + v, r * 2 + s] = 1.0
    w9 = w4t[ky9, kx9, :, :] * v9[:, :, None, None]
    w9 = jnp.pad(w9.transpose(0, 2, 1, 3), ((0, 0), (0, 0), (0, 0), (0, 1)))
    w9 = w9.reshape(9, 256, 16)
    b16 = jnp.tile(jnp.pad(dec_b2, (0, 1)), 4).reshape(1, 16)
    out64 = _dct2(dph, w9, b16)

    xh = out64.reshape(B, 56, 56, 2, 2, 2, 2, 4)
    xh = xh.transpose(0, 1, 3, 5, 2, 4, 6, 7)
    xh = xh.reshape(B, 224, 224, 4)[..., :3]
    xh = xh.transpose(0, 3, 1, 2)

    xsum = _sqdiff_sum(xh, x)
    vq_loss = 1.25 * (qsum / (25088.0 * 256.0)) + xsum / 1204224.0
    return (xh, vq_loss, idx.reshape(B, 56, 56))
